# baseline (speedup 1.0000x reference)
import functools

import jax
import jax.numpy as jnp
from jax import lax
from jax.experimental import pallas as pl
from jax.experimental.pallas import tpu as pltpu
from jax.experimental.pallas import tpu_sc as plsc

NCLU = 512
NPTS = 16 * 64 * 64
NW = 32
PPW = NPTS // NW
LANES = 16
PV = PPW // LANES
PBLK = 4
NBLK = PV // PBLK


def _make_sc_argmin():
    mesh = plsc.VectorSubcoreMesh(core_axis_name="c", subcore_axis_name="s")

    @functools.partial(
        pl.kernel,
        out_type=jax.ShapeDtypeStruct((NPTS,), jnp.int32),
        mesh=mesh,
        scratch_types=[
            pltpu.VMEM((PPW,), jnp.float32),
            pltpu.VMEM((PPW,), jnp.float32),
            pltpu.VMEM((PPW,), jnp.float32),
            pltpu.VMEM((4 * LANES * NCLU,), jnp.float32),
            pltpu.VMEM((PPW,), jnp.int32),
            pltpu.SemaphoreType.DMA,
        ],
    )
    def sc_argmin(xf_hbm, cb_hbm, out_hbm, xs0, xs1, xs2, cbv, outv, sem):
        wid = lax.axis_index("s") * 2 + lax.axis_index("c")
        b = wid // 2
        half = wid % 2
        xoff = b * (3 * 4096) + half * 2048
        pltpu.sync_copy(xf_hbm.at[pl.ds(xoff, PPW)], xs0)
        pltpu.sync_copy(xf_hbm.at[pl.ds(xoff + 4096, PPW)], xs1)
        pltpu.sync_copy(xf_hbm.at[pl.ds(xoff + 8192, PPW)], xs2)
        pltpu.sync_copy(cb_hbm, cbv)

        inf = jnp.full((LANES,), jnp.inf, jnp.float32)
        zero_i = jnp.full((LANES,), 0, jnp.int32)

        def block(blk, _):
            pbase = blk * (PBLK * LANES)
            x0 = [xs0[pl.ds(pbase + p * LANES, LANES)] for p in range(PBLK)]
            x1 = [xs1[pl.ds(pbase + p * LANES, LANES)] for p in range(PBLK)]
            x2 = [xs2[pl.ds(pbase + p * LANES, LANES)] for p in range(PBLK)]

            def cluster(kk, st):
                best, bidx = st
                cb = kk * (4 * LANES)
                m0 = cbv[pl.ds(cb, LANES)]
                m1 = cbv[pl.ds(cb + LANES, LANES)]
                m2 = cbv[pl.ds(cb + 2 * LANES, LANES)]
                bb = cbv[pl.ds(cb + 3 * LANES, LANES)]
                kv = zero_i + kk
                nbest, nbidx = [], []
                for p in range(PBLK):
                    d = bb + x2[p] * m2 + x1[p] * m1 + x0[p] * m0
                    m = d < best[p]
                    nbidx.append(jnp.where(m, kv, bidx[p]))
                    nbest.append(jnp.minimum(d, best[p]))
                return tuple(nbest), tuple(nbidx)

            _, bidx = lax.fori_loop(
                0, NCLU, cluster,
                (tuple(inf for _ in range(PBLK)),
                 tuple(zero_i for _ in range(PBLK))))
            for p in range(PBLK):
                outv[pl.ds(pbase + p * LANES, LANES)] = bidx[p]
            return 0
        lax.fori_loop(0, NBLK, block, 0)

        pltpu.sync_copy(outv, out_hbm.at[pl.ds(wid * PPW, PPW)])

    return sc_argmin


_SC_ARGMIN = _make_sc_argmin()


def kernel(x, C):
    bs, c, h, w = x.shape
    xf = x.reshape(-1)
    bb = (C * C).sum(axis=1)
    cb = jnp.concatenate([-2.0 * C, bb[:, None]], axis=1)
    cbs = jnp.broadcast_to(cb[:, :, None], (NCLU, 4, LANES)).reshape(-1)
    a = _SC_ARGMIN(xf, cbs)
    return a.reshape(bs, h * w)
